# Initial kernel scaffold; baseline (speedup 1.0000x reference)
import jax
import jax.numpy as jnp
from jax.experimental import pallas as pl


def kernel(x, means):
    raise NotImplementedError("write your pallas kernel here")


# speedup vs baseline: 1009.6409x; 1009.6409x over previous
import functools

import jax
import jax.numpy as jnp
from jax.experimental import pallas as pl
from jax.experimental.pallas import tpu as pltpu

COMMIT_SCALE = 0.0001


def _fused_kernel(x_ref, means_ref, dists_ref, loss_ref, *, num_clusters):
    i = pl.program_id(0)
    j = pl.program_id(1)

    x = x_ref[...]
    m = means_ref[...]

    sq = jnp.sum(x * x, axis=-1, keepdims=True)
    norm = jnp.maximum(jnp.sqrt(sq), 1e-12)
    xn = x / norm

    d = jax.lax.dot_general(
        xn, m, (((1,), (1,)), ((), ())), preferred_element_type=jnp.float32
    )
    dists_ref[...] = d

    xnsq = jnp.sum(xn * xn, axis=-1)
    dmax = jnp.max(d, axis=-1)
    iota = jax.lax.broadcasted_iota(jnp.int32, d.shape, 1)
    idx = jnp.min(
        jnp.where(d == dmax[:, None], iota, num_clusters), axis=-1
    )
    msq = jnp.sum(m * m, axis=-1)
    msq_sel = jnp.sum(
        jnp.where(iota == idx[:, None], msq[None, :], 0.0), axis=-1
    )
    partial = jnp.sum(xnsq - 2.0 * dmax + msq_sel)

    @pl.when((i == 0) & (j == 0))
    def _init():
        loss_ref[0, 0] = 0.0

    loss_ref[0, 0] += partial


def kernel(x, means):
    B, H, L, D = x.shape
    Hm, C, Dm = means.shape
    xr = x.reshape(B * H, L, D)

    LT = 512
    grid = (B * H, L // LT)

    dists, loss = pl.pallas_call(
        functools.partial(_fused_kernel, num_clusters=C),
        grid=grid,
        in_specs=[
            pl.BlockSpec((None, LT, D), lambda i, j: (i, j, 0)),
            pl.BlockSpec((None, C, Dm), lambda i, j: (i % Hm, 0, 0)),
        ],
        out_specs=[
            pl.BlockSpec((None, LT, C), lambda i, j: (i, j, 0)),
            pl.BlockSpec(memory_space=pltpu.SMEM),
        ],
        out_shape=[
            jax.ShapeDtypeStruct((B * H, L, C), jnp.float32),
            jax.ShapeDtypeStruct((1, 1), jnp.float32),
        ],
    )(xr, means)

    loss_scalar = loss[0, 0] * (COMMIT_SCALE / (B * H * L * D))
    return (dists.reshape(B, H, L, C), loss_scalar)
